# baseline (speedup 1.0000x reference)
import functools

import jax
import jax.numpy as jnp
import numpy as np
from jax.experimental import pallas as pl

B = 8
P = 2048
NUM_CLASSES = 13


def _mlp_j(layers, x, mask=None, norm=True):
    n = len(layers)
    for i, p in enumerate(layers):
        x = jnp.matmul(x, p['w']) + p['b']
        if i < n - 1:
            if norm:
                red = tuple(range(x.ndim - 1))
                if mask is None:
                    m = jnp.mean(x, axis=red)
                    v = jnp.mean((x - m) ** 2, axis=red)
                else:
                    cnt = jnp.sum(mask) + 1e-6
                    m = jnp.sum(x * mask, axis=red) / cnt
                    v = jnp.sum(((x - m) ** 2) * mask, axis=red) / cnt
                x = (x - m) / jnp.sqrt(v + 1e-5) * p['gamma'] + p['beta']
            x = jax.nn.relu(x)
            if mask is not None:
                x = x * mask
    return x


def _fps_j(pos, S):
    d0 = jnp.sum((pos - pos[0]) ** 2, axis=-1)
    idxs = jnp.zeros((S,), jnp.int32)

    def body(i, st):
        dists, ids = st
        nxt = jnp.argmax(dists).astype(jnp.int32)
        ids = ids.at[i].set(nxt)
        d = jnp.sum((pos - pos[nxt]) ** 2, axis=-1)
        return (jnp.minimum(dists, d), ids)

    dists, idxs = jax.lax.fori_loop(1, S, body, (d0, idxs))
    return idxs


def _radius_j(pos, qpos, r, max_n=64):
    d2 = jnp.sum((qpos[:, :, None, :] - pos[:, None, :, :]) ** 2, axis=-1)
    d2m = jnp.where(d2 <= r * r, d2, jnp.inf)
    negd, idx = jax.lax.top_k(-d2m, max_n)
    valid = jnp.isfinite(negd)
    return idx, valid


def _sa_j(layers, x, pos, ratio, r):
    Pn = pos.shape[1]
    S = int(Pn * ratio)
    sidx = jax.vmap(lambda p: _fps_j(p, S))(pos)
    qpos = jnp.take_along_axis(pos, sidx[:, :, None], axis=1)
    nidx, valid = _radius_j(pos, qpos, r, 64)
    pj = jax.vmap(lambda p, i: p[i])(pos, nidx)
    xj = jax.vmap(lambda xx, i: xx[i])(x, nidx)
    msg = jnp.concatenate([xj, pj - qpos[:, :, None, :]], axis=-1)
    mask = valid[..., None].astype(jnp.float32)
    h = _mlp_j(layers, msg, mask=mask, norm=True)
    h = jnp.where(valid[..., None], h, -jnp.inf)
    out = jnp.max(h, axis=2)
    out = jnp.where(jnp.isfinite(out), out, 0.0)
    return out, qpos


def _knn_j(x, pos, pos_skip, k):
    d2 = jnp.sum((pos_skip[:, :, None, :] - pos[:, None, :, :]) ** 2, axis=-1)
    kk = min(k, pos.shape[1])
    _, idx = jax.lax.top_k(-d2, kk)
    d2k = jnp.take_along_axis(d2, idx, axis=2)
    w = 1.0 / jnp.maximum(d2k, 1e-16)
    xk = jax.vmap(lambda xx, ii: xx[ii])(x, idx)
    return jnp.sum(xk * w[..., None], axis=2) / jnp.sum(w, axis=2, keepdims=True)


def _head_body(y_ref, w1_ref, b1_ref, w2_ref, b2_ref, o_ref):
    h = jnp.maximum(
        jnp.dot(y_ref[...], w1_ref[...], preferred_element_type=jnp.float32)
        + b1_ref[...], 0.0)
    o_ref[...] = (
        jnp.dot(h, w2_ref[...], preferred_element_type=jnp.float32)
        + b2_ref[...])


def _head_pallas(y, head_params):
    n, _ = y.shape
    w1 = head_params[0]['w']
    b1 = head_params[0]['b'].reshape(1, -1)
    w2 = head_params[1]['w']
    b2 = head_params[1]['b'].reshape(1, -1)
    out = pl.pallas_call(
        _head_body,
        out_shape=jax.ShapeDtypeStruct((n, w2.shape[1]), jnp.float32),
    )(y, w1, b1, w2, b2)
    return out


def kernel(x, pos, batch, params):
    xb = x[:, :6].reshape(B, P, 6)
    pb = pos.reshape(B, P, 3)
    x1, p1 = _sa_j(params['sa1'], xb, pb, 0.25, 0.2)
    x2, p2 = _sa_j(params['sa2'], x1, p1, 0.25, 0.4)
    x3, p3 = _sa_j(params['sa3'], x2, p2, 0.25, 0.8)
    h4 = _mlp_j(params['sa4'], jnp.concatenate([x3, p3], axis=-1), norm=True)
    x4 = jnp.max(h4, axis=1, keepdims=True)
    p4 = jnp.zeros((B, 1, 3), jnp.float32)
    y = _knn_j(x4, p4, p3, 1)
    y = _mlp_j(params['fp4'], jnp.concatenate([y, x3], axis=-1), norm=True)
    y = _knn_j(y, p3, p2, 3)
    y = _mlp_j(params['fp3'], jnp.concatenate([y, x2], axis=-1), norm=True)
    y = _knn_j(y, p2, p1, 3)
    y = _mlp_j(params['fp2'], jnp.concatenate([y, x1], axis=-1), norm=True)
    y = _knn_j(y, p1, pb, 3)
    y = _mlp_j(params['fp1'], jnp.concatenate([y, xb], axis=-1), norm=True)
    out = _head_pallas(y.reshape(B * P, -1), params['head'])
    return out.reshape(B * P, NUM_CLASSES)


# speedup vs baseline: 1.2880x; 1.2880x over previous
import functools

import jax
import jax.numpy as jnp
import numpy as np
from jax.experimental import pallas as pl

B = 8
P = 2048
NUM_CLASSES = 13


def _mlp_j(layers, x, mask=None, norm=True):
    n = len(layers)
    for i, p in enumerate(layers):
        x = jnp.matmul(x, p['w']) + p['b']
        if i < n - 1:
            if norm:
                red = tuple(range(x.ndim - 1))
                if mask is None:
                    m = jnp.mean(x, axis=red)
                    v = jnp.mean((x - m) ** 2, axis=red)
                else:
                    cnt = jnp.sum(mask) + 1e-6
                    m = jnp.sum(x * mask, axis=red) / cnt
                    v = jnp.sum(((x - m) ** 2) * mask, axis=red) / cnt
                x = (x - m) / jnp.sqrt(v + 1e-5) * p['gamma'] + p['beta']
            x = jax.nn.relu(x)
            if mask is not None:
                x = x * mask
    return x


def _fps_j(pos, S):
    d0 = jnp.sum((pos - pos[0]) ** 2, axis=-1)
    idxs = jnp.zeros((S,), jnp.int32)

    def body(i, st):
        dists, ids = st
        nxt = jnp.argmax(dists).astype(jnp.int32)
        ids = ids.at[i].set(nxt)
        d = jnp.sum((pos - pos[nxt]) ** 2, axis=-1)
        return (jnp.minimum(dists, d), ids)

    dists, idxs = jax.lax.fori_loop(1, S, body, (d0, idxs))
    return idxs


def _radius_j(pos, qpos, r, max_n=64):
    d2 = jnp.sum((qpos[:, :, None, :] - pos[:, None, :, :]) ** 2, axis=-1)
    d2m = jnp.where(d2 <= r * r, d2, jnp.inf)
    negd, idx = jax.lax.top_k(-d2m, max_n)
    valid = jnp.isfinite(negd)
    return idx, valid


def _sa_j(layers, x, pos, qpos, nidx, valid):
    pj = jax.vmap(lambda p, i: p[i])(pos, nidx)
    xj = jax.vmap(lambda xx, i: xx[i])(x, nidx)
    msg = jnp.concatenate([xj, pj - qpos[:, :, None, :]], axis=-1)
    mask = valid[..., None].astype(jnp.float32)
    h = _mlp_j(layers, msg, mask=mask, norm=True)
    h = jnp.where(valid[..., None], h, -jnp.inf)
    out = jnp.max(h, axis=2)
    out = jnp.where(jnp.isfinite(out), out, 0.0)
    return out


def _knn_j(x, pos, pos_skip, k):
    d2 = jnp.sum((pos_skip[:, :, None, :] - pos[:, None, :, :]) ** 2, axis=-1)
    kk = min(k, pos.shape[1])
    _, idx = jax.lax.top_k(-d2, kk)
    d2k = jnp.take_along_axis(d2, idx, axis=2)
    w = 1.0 / jnp.maximum(d2k, 1e-16)
    xk = jax.vmap(lambda xx, ii: xx[ii])(x, idx)
    return jnp.sum(xk * w[..., None], axis=2) / jnp.sum(w, axis=2, keepdims=True)


def _fps_stage(px, py, pz, S):
    Bn, N = px.shape
    il = jax.lax.broadcasted_iota(jnp.int32, (Bn, N), 1)
    ilS = jax.lax.broadcasted_iota(jnp.int32, (Bn, S), 1)
    c0x, c0y, c0z = px[:, :1], py[:, :1], pz[:, :1]
    d = (px - c0x) ** 2 + (py - c0y) ** 2 + (pz - c0z) ** 2
    first = (ilS == 0).astype(jnp.float32)
    qx = c0x * first
    qy = c0y * first
    qz = c0z * first

    def body(i, st):
        d, qx, qy, qz = st
        m = jnp.max(d, axis=1, keepdims=True)
        nxt = jnp.min(jnp.where(d == m, il, N), axis=1, keepdims=True)
        sel = (il == nxt).astype(jnp.float32)
        cx = jnp.sum(px * sel, axis=1, keepdims=True)
        cy = jnp.sum(py * sel, axis=1, keepdims=True)
        cz = jnp.sum(pz * sel, axis=1, keepdims=True)
        dn = (px - cx) ** 2 + (py - cy) ** 2 + (pz - cz) ** 2
        d = jnp.minimum(d, dn)
        hit = (ilS == i).astype(jnp.float32)
        return (d, qx + cx * hit, qy + cy * hit, qz + cz * hit)

    _, qx, qy, qz = jax.lax.fori_loop(1, S, body, (d, qx, qy, qz))
    return qx, qy, qz


def _geometry_body(px_ref, py_ref, pz_ref,
                   q1x_ref, q1y_ref, q1z_ref,
                   q2x_ref, q2y_ref, q2z_ref,
                   q3x_ref, q3y_ref, q3z_ref):
    q1 = _fps_stage(px_ref[...], py_ref[...], pz_ref[...], 512)
    q1x_ref[...], q1y_ref[...], q1z_ref[...] = q1
    q2 = _fps_stage(*q1, 128)
    q2x_ref[...], q2y_ref[...], q2z_ref[...] = q2
    q3 = _fps_stage(*q2, 32)
    q3x_ref[...], q3y_ref[...], q3z_ref[...] = q3


def _geometry_pallas(pb):
    px = pb[:, :, 0]
    py = pb[:, :, 1]
    pz = pb[:, :, 2]
    f = jnp.float32
    outs = pl.pallas_call(
        _geometry_body,
        out_shape=[jax.ShapeDtypeStruct((B, s), f)
                   for s in (512, 512, 512, 128, 128, 128, 32, 32, 32)],
    )(px, py, pz)
    q1 = jnp.stack(outs[0:3], axis=-1)
    q2 = jnp.stack(outs[3:6], axis=-1)
    q3 = jnp.stack(outs[6:9], axis=-1)
    return q1, q2, q3


def _select_body(q_ref, p_ref, nidx_ref, valid_ref, *, r2, S, N):
    q = q_ref[0]
    pT = p_ref[0]
    qx, qy, qz = q[:, 0:1], q[:, 1:2], q[:, 2:3]
    px, py, pz = pT[0:1, :], pT[1:2, :], pT[2:3, :]
    d2 = (qx - px) ** 2 + (qy - py) ** 2 + (qz - pz) ** 2
    il = jax.lax.broadcasted_iota(jnp.int32, (S, N), 1)
    ilK = jax.lax.broadcasted_iota(jnp.int32, (S, 64), 1)
    inf = jnp.float32(jnp.inf)

    def body(j, st):
        d2, nidx, valid = st
        m = jnp.min(d2, axis=1, keepdims=True)
        e = jnp.min(jnp.where(d2 == m, il, N), axis=1, keepdims=True)
        d2 = jnp.where(il == e, inf, d2)
        hit = (ilK == j)
        nidx = jnp.where(hit, e, nidx)
        valid = jnp.where(hit, (m <= r2).astype(jnp.float32), valid)
        return (d2, nidx, valid)

    nidx0 = jnp.zeros((S, 64), jnp.int32)
    valid0 = jnp.zeros((S, 64), jnp.float32)
    _, nidx, valid = jax.lax.fori_loop(0, 64, body, (d2, nidx0, valid0))
    nidx_ref[0] = nidx
    valid_ref[0] = valid


def _select_pallas(qpos, pos, r):
    Bn, S, _ = qpos.shape
    N = pos.shape[1]
    posT = jnp.swapaxes(pos, 1, 2)
    body = functools.partial(_select_body, r2=np.float32(r * r), S=S, N=N)
    nidx, valid = pl.pallas_call(
        body,
        grid=(Bn,),
        in_specs=[
            pl.BlockSpec((1, S, 3), lambda b: (b, 0, 0)),
            pl.BlockSpec((1, 3, N), lambda b: (b, 0, 0)),
        ],
        out_specs=[
            pl.BlockSpec((1, S, 64), lambda b: (b, 0, 0)),
            pl.BlockSpec((1, S, 64), lambda b: (b, 0, 0)),
        ],
        out_shape=[
            jax.ShapeDtypeStruct((Bn, S, 64), jnp.int32),
            jax.ShapeDtypeStruct((Bn, S, 64), jnp.float32),
        ],
    )(qpos, posT)
    return nidx, valid > 0.5


def _head_body(y_ref, w1_ref, b1_ref, w2_ref, b2_ref, o_ref):
    h = jnp.maximum(
        jnp.dot(y_ref[...], w1_ref[...], preferred_element_type=jnp.float32)
        + b1_ref[...], 0.0)
    o_ref[...] = (
        jnp.dot(h, w2_ref[...], preferred_element_type=jnp.float32)
        + b2_ref[...])


def _head_pallas(y, head_params):
    n, _ = y.shape
    w1 = head_params[0]['w']
    b1 = head_params[0]['b'].reshape(1, -1)
    w2 = head_params[1]['w']
    b2 = head_params[1]['b'].reshape(1, -1)
    out = pl.pallas_call(
        _head_body,
        out_shape=jax.ShapeDtypeStruct((n, w2.shape[1]), jnp.float32),
    )(y, w1, b1, w2, b2)
    return out


def kernel(x, pos, batch, params):
    xb = x[:, :6].reshape(B, P, 6)
    pb = pos.reshape(B, P, 3)
    p1, p2, p3 = _geometry_pallas(pb)
    n1, v1 = _select_pallas(p1, pb, 0.2)
    n2, v2 = _select_pallas(p2, p1, 0.4)
    n3, v3 = _select_pallas(p3, p2, 0.8)
    x1 = _sa_j(params['sa1'], xb, pb, p1, n1, v1)
    x2 = _sa_j(params['sa2'], x1, p1, p2, n2, v2)
    x3 = _sa_j(params['sa3'], x2, p2, p3, n3, v3)
    h4 = _mlp_j(params['sa4'], jnp.concatenate([x3, p3], axis=-1), norm=True)
    x4 = jnp.max(h4, axis=1, keepdims=True)
    p4 = jnp.zeros((B, 1, 3), jnp.float32)
    y = _knn_j(x4, p4, p3, 1)
    y = _mlp_j(params['fp4'], jnp.concatenate([y, x3], axis=-1), norm=True)
    y = _knn_j(y, p3, p2, 3)
    y = _mlp_j(params['fp3'], jnp.concatenate([y, x2], axis=-1), norm=True)
    y = _knn_j(y, p2, p1, 3)
    y = _mlp_j(params['fp2'], jnp.concatenate([y, x1], axis=-1), norm=True)
    y = _knn_j(y, p1, pb, 3)
    y = _mlp_j(params['fp1'], jnp.concatenate([y, xb], axis=-1), norm=True)
    out = _head_pallas(y.reshape(B * P, -1), params['head'])
    return out.reshape(B * P, NUM_CLASSES)


# speedup vs baseline: 1.2883x; 1.0002x over previous
import functools

import jax
import jax.numpy as jnp
import numpy as np
from jax.experimental import pallas as pl
from jax.experimental.pallas import tpu as pltpu
from jax.experimental.pallas import tpu_sc as plsc

B = 8
P = 2048
NUM_CLASSES = 13


def _mlp_j(layers, x, mask=None, norm=True):
    n = len(layers)
    for i, p in enumerate(layers):
        x = jnp.matmul(x, p['w']) + p['b']
        if i < n - 1:
            if norm:
                red = tuple(range(x.ndim - 1))
                if mask is None:
                    m = jnp.mean(x, axis=red)
                    v = jnp.mean((x - m) ** 2, axis=red)
                else:
                    cnt = jnp.sum(mask) + 1e-6
                    m = jnp.sum(x * mask, axis=red) / cnt
                    v = jnp.sum(((x - m) ** 2) * mask, axis=red) / cnt
                x = (x - m) / jnp.sqrt(v + 1e-5) * p['gamma'] + p['beta']
            x = jax.nn.relu(x)
            if mask is not None:
                x = x * mask
    return x


def _fps_j(pos, S):
    d0 = jnp.sum((pos - pos[0]) ** 2, axis=-1)
    idxs = jnp.zeros((S,), jnp.int32)

    def body(i, st):
        dists, ids = st
        nxt = jnp.argmax(dists).astype(jnp.int32)
        ids = ids.at[i].set(nxt)
        d = jnp.sum((pos - pos[nxt]) ** 2, axis=-1)
        return (jnp.minimum(dists, d), ids)

    dists, idxs = jax.lax.fori_loop(1, S, body, (d0, idxs))
    return idxs


def _radius_j(pos, qpos, r, max_n=64):
    d2 = jnp.sum((qpos[:, :, None, :] - pos[:, None, :, :]) ** 2, axis=-1)
    d2m = jnp.where(d2 <= r * r, d2, jnp.inf)
    negd, idx = jax.lax.top_k(-d2m, max_n)
    valid = jnp.isfinite(negd)
    return idx, valid


def _sa_j(layers, x, pos, qpos, nidx, valid):
    pj = jax.vmap(lambda p, i: p[i])(pos, nidx)
    xj = jax.vmap(lambda xx, i: xx[i])(x, nidx)
    msg = jnp.concatenate([xj, pj - qpos[:, :, None, :]], axis=-1)
    mask = valid[..., None].astype(jnp.float32)
    h = _mlp_j(layers, msg, mask=mask, norm=True)
    h = jnp.where(valid[..., None], h, -jnp.inf)
    out = jnp.max(h, axis=2)
    out = jnp.where(jnp.isfinite(out), out, 0.0)
    return out


def _knn_j(x, pos, pos_skip, k):
    d2 = jnp.sum((pos_skip[:, :, None, :] - pos[:, None, :, :]) ** 2, axis=-1)
    kk = min(k, pos.shape[1])
    _, idx = jax.lax.top_k(-d2, kk)
    d2k = jnp.take_along_axis(d2, idx, axis=2)
    w = 1.0 / jnp.maximum(d2k, 1e-16)
    xk = jax.vmap(lambda xx, ii: xx[ii])(x, idx)
    return jnp.sum(xk * w[..., None], axis=2) / jnp.sum(w, axis=2, keepdims=True)


def _fps_stage(px, py, pz, S):
    Bn, N = px.shape
    il = jax.lax.broadcasted_iota(jnp.int32, (Bn, N), 1)
    ilS = jax.lax.broadcasted_iota(jnp.int32, (Bn, S), 1)
    c0x, c0y, c0z = px[:, :1], py[:, :1], pz[:, :1]
    d = (px - c0x) ** 2 + (py - c0y) ** 2 + (pz - c0z) ** 2
    first = (ilS == 0).astype(jnp.float32)
    qx = c0x * first
    qy = c0y * first
    qz = c0z * first

    def body(i, st):
        d, qx, qy, qz = st
        m = jnp.max(d, axis=1, keepdims=True)
        nxt = jnp.min(jnp.where(d == m, il, N), axis=1, keepdims=True)
        sel = (il == nxt).astype(jnp.float32)
        cx = jnp.sum(px * sel, axis=1, keepdims=True)
        cy = jnp.sum(py * sel, axis=1, keepdims=True)
        cz = jnp.sum(pz * sel, axis=1, keepdims=True)
        dn = (px - cx) ** 2 + (py - cy) ** 2 + (pz - cz) ** 2
        d = jnp.minimum(d, dn)
        hit = (ilS == i).astype(jnp.float32)
        return (d, qx + cx * hit, qy + cy * hit, qz + cz * hit)

    _, qx, qy, qz = jax.lax.fori_loop(1, S, body, (d, qx, qy, qz))
    return qx, qy, qz


def _geometry_body(px_ref, py_ref, pz_ref,
                   q1x_ref, q1y_ref, q1z_ref,
                   q2x_ref, q2y_ref, q2z_ref,
                   q3x_ref, q3y_ref, q3z_ref):
    q1 = _fps_stage(px_ref[...], py_ref[...], pz_ref[...], 512)
    q1x_ref[...], q1y_ref[...], q1z_ref[...] = q1
    q2 = _fps_stage(*q1, 128)
    q2x_ref[...], q2y_ref[...], q2z_ref[...] = q2
    q3 = _fps_stage(*q2, 32)
    q3x_ref[...], q3y_ref[...], q3z_ref[...] = q3


def _geometry_pallas(pb):
    px = pb[:, :, 0]
    py = pb[:, :, 1]
    pz = pb[:, :, 2]
    f = jnp.float32
    outs = pl.pallas_call(
        _geometry_body,
        out_shape=[jax.ShapeDtypeStruct((B, s), f)
                   for s in (512, 512, 512, 128, 128, 128, 32, 32, 32)],
    )(px, py, pz)
    q1 = jnp.stack(outs[0:3], axis=-1)
    q2 = jnp.stack(outs[3:6], axis=-1)
    q3 = jnp.stack(outs[6:9], axis=-1)
    return q1, q2, q3


def _select_body(q_ref, p_ref, nidx_ref, valid_ref, *, r2, S, N):
    q = q_ref[0]
    pT = p_ref[0]
    qx, qy, qz = q[:, 0:1], q[:, 1:2], q[:, 2:3]
    px, py, pz = pT[0:1, :], pT[1:2, :], pT[2:3, :]
    d2 = (qx - px) ** 2 + (qy - py) ** 2 + (qz - pz) ** 2
    il = jax.lax.broadcasted_iota(jnp.int32, (S, N), 1)
    ilK = jax.lax.broadcasted_iota(jnp.int32, (S, 64), 1)
    inf = jnp.float32(jnp.inf)

    def body(j, st):
        d2, nidx, valid = st
        m = jnp.min(d2, axis=1, keepdims=True)
        e = jnp.min(jnp.where(d2 == m, il, N), axis=1, keepdims=True)
        d2 = jnp.where(il == e, inf, d2)
        hit = (ilK == j)
        nidx = jnp.where(hit, e, nidx)
        valid = jnp.where(hit, (m <= r2).astype(jnp.float32), valid)
        return (d2, nidx, valid)

    nidx0 = jnp.zeros((S, 64), jnp.int32)
    valid0 = jnp.zeros((S, 64), jnp.float32)
    _, nidx, valid = jax.lax.fori_loop(0, 64, body, (d2, nidx0, valid0))
    nidx_ref[0] = nidx + pl.program_id(0) * N
    valid_ref[0] = valid


def _select_pallas(qpos, pos, r):
    Bn, S, _ = qpos.shape
    N = pos.shape[1]
    posT = jnp.swapaxes(pos, 1, 2)
    body = functools.partial(_select_body, r2=np.float32(r * r), S=S, N=N)
    nidx, valid = pl.pallas_call(
        body,
        grid=(Bn,),
        in_specs=[
            pl.BlockSpec((1, S, 3), lambda b: (b, 0, 0)),
            pl.BlockSpec((1, 3, N), lambda b: (b, 0, 0)),
        ],
        out_specs=[
            pl.BlockSpec((1, S, 64), lambda b: (b, 0, 0)),
            pl.BlockSpec((1, S, 64), lambda b: (b, 0, 0)),
        ],
        out_shape=[
            jax.ShapeDtypeStruct((Bn, S, 64), jnp.int32),
            jax.ShapeDtypeStruct((Bn, S, 64), jnp.float32),
        ],
    )(qpos, posT)
    return nidx.reshape(1, Bn * S * 64), valid


_SC_WINDOW = 128


def _sc_gather(table, idx):
    ni = idx.shape[1]
    cdim = table.shape[1]
    mesh = plsc.VectorSubcoreMesh(core_axis_name="c", subcore_axis_name="s")

    @functools.partial(
        pl.kernel,
        out_type=jax.ShapeDtypeStruct((ni, cdim), table.dtype),
        mesh=mesh,
    )
    def gather_kernel(tab_hbm, idx_hbm, out_hbm):
        def body(i_vmem, o_vmem):
            pltpu.sync_copy(tab_hbm.at[i_vmem.at[0]], o_vmem)

        pltpu.emit_pipeline(
            body,
            grid=(ni // _SC_WINDOW,),
            in_specs=[pl.BlockSpec((1, _SC_WINDOW), index_map=lambda i: (0, i))],
            out_specs=[pl.BlockSpec((_SC_WINDOW, cdim),
                                    index_map=lambda i: (i, 0))],
            core_axis_name=("c", "s"),
            dimension_semantics=(pltpu.PARALLEL,),
        )(idx_hbm, out_hbm)

    return gather_kernel(table, idx)


def _z1_of(g_ref, qp_ref, w1_ref, b1_ref, S, CP, C1):
    msg = g_ref[...][:, :CP].reshape(S, 64, CP) - qp_ref[0][:, None, :]
    z1 = (jnp.dot(msg.reshape(S * 64, CP), w1_ref[...],
                  preferred_element_type=jnp.float32) + b1_ref[...])
    return z1.reshape(S, 64, C1)


def _pass_a_body(g_ref, qp_ref, w1_ref, b1_ref, z1_ref, *, S, CP, C1):
    z1_ref[...] = _z1_of(g_ref, qp_ref, w1_ref, b1_ref, S, CP, C1).reshape(
        S * 64, C1)


def _norm(z, m_ref, var_ref, gamma_ref, beta_ref):
    zn = ((z - m_ref[...][0][None, None, :])
          / jnp.sqrt(var_ref[...] + 1e-5)[0][None, None, :])
    return jnp.maximum(zn * gamma_ref[...][0][None, None, :]
                       + beta_ref[...][0][None, None, :], 0.0)


def _pass_b_body(z1_ref, v_ref, m_ref, var_ref, gam_ref, bet_ref,
                 w2_ref, b2_ref, z2_ref, *, S, C1, C2):
    v = v_ref[0]
    z1 = z1_ref[...].reshape(S, 64, C1)
    z1n = _norm(z1, m_ref, var_ref, gam_ref, bet_ref) * v[:, :, None]
    z2_ref[...] = (jnp.dot(z1n.reshape(S * 64, C1), w2_ref[...],
                           preferred_element_type=jnp.float32)
                   + b2_ref[...])


def _pass_c_body(z2_ref, v_ref, m_ref, var_ref, gam_ref, bet_ref,
                 w3_ref, b3_ref, out_ref, *, S, C2, C3):
    v = v_ref[0]
    z2 = z2_ref[...].reshape(S, 64, C2)
    z2n = _norm(z2, m_ref, var_ref, gam_ref, bet_ref) * v[:, :, None]
    z3 = (jnp.dot(z2n.reshape(S * 64, C2), w3_ref[...],
                  preferred_element_type=jnp.float32)
          + b3_ref[...]).reshape(S, 64, C3)
    h = jnp.where(v[:, :, None] > 0.5, z3, -jnp.inf)
    mx = jnp.max(h, axis=1)
    out_ref[0] = jnp.where(jnp.isfinite(mx), mx, 0.0)


def _masked_stats(z, maskf):
    cnt = jnp.sum(maskf) + 1e-6
    m = jnp.sum(z * maskf, axis=(0, 1, 2)) / cnt
    v = jnp.sum(((z - m) ** 2) * maskf, axis=(0, 1, 2)) / cnt
    return m.reshape(1, -1), v.reshape(1, -1)


def _sa_fast(layers, x, pos, qpos, nidx_flat, valid):
    Bn, N, C = x.shape
    S = qpos.shape[1]
    CP = C + 3
    PW = ((CP + 127) // 128) * 128
    xp = jnp.concatenate([x, pos], axis=-1).reshape(Bn * N, CP)
    tbl = jnp.pad(xp, ((0, 0), (0, PW - CP)))
    g = _sc_gather(tbl, nidx_flat).reshape(Bn, S, 64, PW)
    xj = g[..., :C]
    pj = g[..., C:CP]
    msg = jnp.concatenate([xj, pj - qpos[:, :, None, :]], axis=-1)
    vb = valid > 0.5
    mask = vb[..., None].astype(jnp.float32)
    h = _mlp_j(layers, msg, mask=mask, norm=True)
    h = jnp.where(vb[..., None], h, -jnp.inf)
    out = jnp.max(h, axis=2)
    return jnp.where(jnp.isfinite(out), out, 0.0)


def _sa_pallas(layers, x, pos, qpos, nidx_flat, valid):
    Bn, N, C = x.shape
    S = qpos.shape[1]
    C1 = layers[0]['w'].shape[1]
    C2 = layers[1]['w'].shape[1]
    C3 = layers[2]['w'].shape[1]
    f = jnp.float32
    CP = C + 3
    PW = ((CP + 127) // 128) * 128
    xp = jnp.concatenate([x, pos], axis=-1).reshape(Bn * N, CP)
    tbl = jnp.pad(xp, ((0, 0), (0, PW - CP)))
    qpad = jnp.pad(qpos, ((0, 0), (0, 0), (C, 0)))
    w1 = layers[0]['w']
    b1 = layers[0]['b'].reshape(1, C1)

    g = _sc_gather(tbl, nidx_flat)

    ch = min(S, 128)
    nch = S // ch
    row_spec = lambda c: pl.BlockSpec(
        (ch * 64, c), lambda b, sc: (b * nch + sc, 0))
    q_spec = lambda c: pl.BlockSpec((1, ch, c), lambda b, sc: (b, sc, 0))
    full_spec = lambda r, c: pl.BlockSpec((r, c), lambda b, sc: (0, 0))
    z1 = pl.pallas_call(
        functools.partial(_pass_a_body, S=ch, CP=CP, C1=C1),
        grid=(Bn, nch),
        in_specs=[row_spec(PW), q_spec(CP),
                  full_spec(CP, C1), full_spec(1, C1)],
        out_specs=row_spec(C1),
        out_shape=jax.ShapeDtypeStruct((Bn * S * 64, C1), f),
    )(g, qpad, w1, b1)

    maskf = (valid > 0.5)[..., None].astype(f)
    m1, var1 = _masked_stats(z1.reshape(Bn, S, 64, C1), maskf)

    z2 = pl.pallas_call(
        functools.partial(_pass_b_body, S=ch, C1=C1, C2=C2),
        grid=(Bn, nch),
        in_specs=[row_spec(C1), q_spec(64),
                  full_spec(1, C1), full_spec(1, C1),
                  full_spec(1, C1), full_spec(1, C1),
                  full_spec(C1, C2), full_spec(1, C2)],
        out_specs=row_spec(C2),
        out_shape=jax.ShapeDtypeStruct((Bn * S * 64, C2), f),
    )(z1, valid, m1, var1,
      layers[0]['gamma'].reshape(1, C1), layers[0]['beta'].reshape(1, C1),
      layers[1]['w'], layers[1]['b'].reshape(1, C2))

    m2, var2 = _masked_stats(z2.reshape(Bn, S, 64, C2), maskf)

    out = pl.pallas_call(
        functools.partial(_pass_c_body, S=ch, C2=C2, C3=C3),
        grid=(Bn, nch),
        in_specs=[row_spec(C2), q_spec(64),
                  full_spec(1, C2), full_spec(1, C2),
                  full_spec(1, C2), full_spec(1, C2),
                  full_spec(C2, C3), full_spec(1, C3)],
        out_specs=q_spec(C3),
        out_shape=jax.ShapeDtypeStruct((Bn, S, C3), f),
    )(z2, valid, m2, var2,
      layers[1]['gamma'].reshape(1, C2), layers[1]['beta'].reshape(1, C2),
      layers[2]['w'], layers[2]['b'].reshape(1, C3))
    return out


def _head_body(y_ref, w1_ref, b1_ref, w2_ref, b2_ref, o_ref):
    h = jnp.maximum(
        jnp.dot(y_ref[...], w1_ref[...], preferred_element_type=jnp.float32)
        + b1_ref[...], 0.0)
    o_ref[...] = (
        jnp.dot(h, w2_ref[...], preferred_element_type=jnp.float32)
        + b2_ref[...])


def _head_pallas(y, head_params):
    n, _ = y.shape
    w1 = head_params[0]['w']
    b1 = head_params[0]['b'].reshape(1, -1)
    w2 = head_params[1]['w']
    b2 = head_params[1]['b'].reshape(1, -1)
    out = pl.pallas_call(
        _head_body,
        out_shape=jax.ShapeDtypeStruct((n, w2.shape[1]), jnp.float32),
    )(y, w1, b1, w2, b2)
    return out


def kernel(x, pos, batch, params):
    xb = x[:, :6].reshape(B, P, 6)
    pb = pos.reshape(B, P, 3)
    p1, p2, p3 = _geometry_pallas(pb)
    n1, v1 = _select_pallas(p1, pb, 0.2)
    n2, v2 = _select_pallas(p2, p1, 0.4)
    n3, v3 = _select_pallas(p3, p2, 0.8)
    n1l = n1.reshape(B, 512, 64) - (jnp.arange(B, dtype=jnp.int32) * P)[:, None, None]
    n2l = n2.reshape(B, 128, 64) - (jnp.arange(B, dtype=jnp.int32) * 512)[:, None, None]
    n3l = n3.reshape(B, 32, 64) - (jnp.arange(B, dtype=jnp.int32) * 128)[:, None, None]
    x1 = _sa_j(params['sa1'], xb, pb, p1, n1l, v1 > 0.5)
    x2 = _sa_j(params['sa2'], x1, p1, p2, n2l, v2 > 0.5)
    x3 = _sa_j(params['sa3'], x2, p2, p3, n3l, v3 > 0.5)
    h4 = _mlp_j(params['sa4'], jnp.concatenate([x3, p3], axis=-1), norm=True)
    x4 = jnp.max(h4, axis=1, keepdims=True)
    p4 = jnp.zeros((B, 1, 3), jnp.float32)
    y = _knn_j(x4, p4, p3, 1)
    y = _mlp_j(params['fp4'], jnp.concatenate([y, x3], axis=-1), norm=True)
    y = _knn_j(y, p3, p2, 3)
    y = _mlp_j(params['fp3'], jnp.concatenate([y, x2], axis=-1), norm=True)
    y = _knn_j(y, p2, p1, 3)
    y = _mlp_j(params['fp2'], jnp.concatenate([y, x1], axis=-1), norm=True)
    y = _knn_j(y, p1, pb, 3)
    y = _mlp_j(params['fp1'], jnp.concatenate([y, xb], axis=-1), norm=True)
    out = _head_pallas(y.reshape(B * P, -1), params['head'])
    return out.reshape(B * P, NUM_CLASSES)
